# baseline (speedup 1.0000x reference)
import functools

import jax
import jax.numpy as jnp
from jax import lax
from jax.experimental import pallas as pl
from jax.experimental.pallas import tpu as pltpu
from jax.experimental.pallas import tpu_sc as plsc

BATCH = 16384
DIM = 64
NC, NS = 2, 16
NW = NC * NS
B_PER_W = BATCH // NW
CHUNK = 128
N_CHUNKS = B_PER_W // CHUNK


def _sc_gather(uidx2, iidx2, gu, gi, mu_b, mi_b):
    mesh = plsc.VectorSubcoreMesh(core_axis_name="c", subcore_axis_name="s")

    @functools.partial(
        pl.kernel,
        out_type=[jax.ShapeDtypeStruct((BATCH, DIM), jnp.float32),
                  jax.ShapeDtypeStruct((BATCH, DIM), jnp.float32),
                  jax.ShapeDtypeStruct((BATCH, DIM), jnp.bfloat16),
                  jax.ShapeDtypeStruct((BATCH, DIM), jnp.bfloat16)],
        mesh=mesh,
        scratch_types=[
            pltpu.VMEM((N_CHUNKS, CHUNK), jnp.int32),
            pltpu.VMEM((N_CHUNKS, CHUNK), jnp.int32),
            pltpu.VMEM((CHUNK, DIM), jnp.float32),
            pltpu.VMEM((CHUNK, DIM), jnp.float32),
            pltpu.VMEM((CHUNK, DIM), jnp.bfloat16),
            pltpu.VMEM((CHUNK, DIM), jnp.bfloat16),
            pltpu.SemaphoreType.DMA,
            pltpu.SemaphoreType.DMA,
        ],
        compiler_params=pltpu.CompilerParams(use_tc_tiling_on_sc=False),
    )
    def k(uidx_hbm, iidx_hbm, gu_hbm, gi_hbm, mu_hbm, mi_hbm,
          gu_out, gi_out, mu_out, mi_out,
          uidx_v, iidx_v, fbuf_a, fbuf_b, bbuf_a, bbuf_b, sem_a, sem_b):
        wid = lax.axis_index("s") * NC + lax.axis_index("c")
        crow = wid * N_CHUNKS
        base = wid * B_PER_W
        pltpu.sync_copy(uidx_hbm.at[pl.ds(crow, N_CHUNKS)], uidx_v)
        pltpu.sync_copy(iidx_hbm.at[pl.ds(crow, N_CHUNKS)], iidx_v)

        jobs = []
        for table, idx_v, out, bufs in (
                (gu_hbm, uidx_v, gu_out, (fbuf_a, fbuf_b)),
                (gi_hbm, iidx_v, gi_out, (fbuf_a, fbuf_b)),
                (mu_hbm, uidx_v, mu_out, (bbuf_a, bbuf_b)),
                (mi_hbm, iidx_v, mi_out, (bbuf_a, bbuf_b))):
            for j in range(N_CHUNKS):
                jobs.append((table, idx_v, out, bufs, j))

        sems = (sem_a, sem_b)
        prev = None
        for n, (table, idx_v, out, bufs, j) in enumerate(jobs):
            s = n % 2
            cp = pltpu.async_copy(table.at[idx_v.at[j]], bufs[s], sems[s])
            if prev is not None:
                p_buf, p_out, p_off, p_cp = prev
                p_cp.wait()
                pltpu.sync_copy(p_buf, p_out.at[pl.ds(p_off, CHUNK)])
            prev = (bufs[s], out, base + j * CHUNK, cp)
        p_buf, p_out, p_off, p_cp = prev
        p_cp.wait()
        pltpu.sync_copy(p_buf, p_out.at[pl.ds(p_off, CHUNK)])

    return k(uidx2, iidx2, gu, gi, mu_b, mi_b)


BM = 2048


def _tc_mlp(gu_rows, gi_rows, mu_rows, mi_rows, W1, b1, W2, b2, Wf, bf):
    def body(gu_ref, gi_ref, mu_ref, mi_ref,
             w1_ref, b1_ref, w2_ref, b2_ref, wf_ref, bf_ref, out_ref):
        gmf = gu_ref[...] * gi_ref[...]
        w1 = w1_ref[...]
        mu = mu_ref[...].astype(jnp.float32)
        mi = mi_ref[...].astype(jnp.float32)
        h = jnp.dot(mu, w1[:DIM], preferred_element_type=jnp.float32)
        h = h + jnp.dot(mi, w1[DIM:], preferred_element_type=jnp.float32)
        h = jnp.maximum(h + b1_ref[...], 0.0)
        h = jnp.maximum(
            jnp.dot(h, w2_ref[...], preferred_element_type=jnp.float32)
            + b2_ref[...], 0.0)
        wf = wf_ref[...]
        pred = (jnp.dot(gmf, wf[:DIM], preferred_element_type=jnp.float32)
                + jnp.dot(h, wf[DIM:], preferred_element_type=jnp.float32)
                + bf_ref[...])
        out_ref[...] = pred

    grid = (BATCH // BM,)
    rows_spec = pl.BlockSpec((BM, DIM), lambda i: (i, 0))
    full = lambda shape: pl.BlockSpec(shape, lambda i: (0,) * len(shape))
    return pl.pallas_call(
        body,
        grid=grid,
        in_specs=[
            rows_spec, rows_spec, rows_spec, rows_spec,
            full((2 * DIM, DIM)), full((1, DIM)),
            full((DIM, 32)), full((1, 32)),
            full((DIM + 32, 1)), full((1, 1)),
        ],
        out_specs=pl.BlockSpec((BM, 1), lambda i: (i, 0)),
        out_shape=jax.ShapeDtypeStruct((BATCH, 1), jnp.float32),
    )(gu_rows, gi_rows, mu_rows, mi_rows, W1, b1, W2, b2, Wf, bf)


def kernel(user_ids, item_ids, gmf_user_w, gmf_item_w, mlp_user_w, mlp_item_w,
           W1, b1, W2, b2, Wf, bf):
    uidx2 = user_ids.astype(jnp.int32).reshape(BATCH // CHUNK, CHUNK)
    iidx2 = item_ids.astype(jnp.int32).reshape(BATCH // CHUNK, CHUNK)
    mu_b = mlp_user_w.astype(jnp.bfloat16)
    mi_b = mlp_item_w.astype(jnp.bfloat16)
    gu, gi, mu, mi = _sc_gather(uidx2, iidx2,
                                gmf_user_w, gmf_item_w, mu_b, mi_b)
    pred = _tc_mlp(gu, gi, mu, mi,
                   W1, b1.reshape(1, DIM), W2, b2.reshape(1, 32),
                   Wf, bf.reshape(1, 1))
    return pred[:, 0]
